# baseline (speedup 1.0000x reference)
import functools

import jax
import jax.numpy as jnp
from jax import lax
from jax.experimental import pallas as pl
from jax.experimental.pallas import tpu as pltpu
from jax.experimental.pallas import tpu_sc as plsc

VOCAB = 1000000
D = 32
CHUNK = 1024
GROUP = 128
LANES = 16


@functools.partial(jax.jit, static_argnames=("n",))
def _lookup(table, idx, n):
    info = plsc.get_sparse_core_info()
    nc, ns = info.num_cores, info.num_subcores
    nw = nc * ns
    per_w = n // nw
    n_chunks = per_w // CHUNK
    mesh = plsc.VectorSubcoreMesh(core_axis_name="c", subcore_axis_name="s")

    @functools.partial(
        pl.kernel,
        mesh=mesh,
        compiler_params=pltpu.CompilerParams(use_tc_tiling_on_sc=False),
        out_type=jax.ShapeDtypeStruct((n, D), jnp.float32),
        scratch_types=[
            pltpu.VMEM((CHUNK,), jnp.int32),
            pltpu.VMEM((CHUNK,), jnp.int32),
            pltpu.VMEM((CHUNK,), jnp.float32),
            pltpu.VMEM((CHUNK, D), jnp.float32),
            pltpu.SemaphoreType.DMA,
        ],
    )
    def k(table_hbm, idx_hbm, out_hbm, idx_raw, idx_safe, maskf, rows, sem):
        wid = lax.axis_index("s") * nc + lax.axis_index("c")
        base0 = wid * per_w

        def chunk_body(i, carry):
            base = base0 + i * CHUNK
            pltpu.sync_copy(idx_hbm.at[pl.ds(base, CHUNK)], idx_raw)

            def fix(v, c):
                iv = idx_raw[pl.ds(v * LANES, LANES)]
                idx_safe[pl.ds(v * LANES, LANES)] = jnp.maximum(iv - VOCAB, 0)
                maskf[pl.ds(v * LANES, LANES)] = jnp.where(
                    iv >= VOCAB, 1.0, 0.0
                ).astype(jnp.float32)
                return c

            lax.fori_loop(0, CHUNK // LANES, fix, 0)

            copies = [
                pltpu.async_copy(
                    table_hbm.at[idx_safe.at[pl.ds(g * GROUP, GROUP)]],
                    rows.at[pl.ds(g * GROUP, GROUP)],
                    sem,
                )
                for g in range(CHUNK // GROUP)
            ]
            for cp in copies:
                cp.wait()

            def zero(v, c):
                mvec = maskf[pl.ds(v * LANES, LANES)]
                for j in range(LANES):
                    r = v * LANES + j
                    m = jnp.full((LANES,), mvec[j], jnp.float32)
                    for h in range(D // LANES):
                        sl = pl.ds(h * LANES, LANES)
                        rows[r, sl] = rows[r, sl] * m
                return c

            lax.fori_loop(0, CHUNK // LANES, zero, 0)

            pltpu.sync_copy(rows, out_hbm.at[pl.ds(base, CHUNK)])
            return carry

        lax.fori_loop(0, n_chunks, chunk_body, 0)

    return k(table, idx)


def kernel(inputs, embeddings):
    b, s = inputs.shape
    n = b * s
    idx = inputs.reshape(n).astype(jnp.int32)
    out = _lookup(embeddings, idx, n)
    return out.reshape(b, s, D)
